# baseline (speedup 1.0000x reference)
import jax
import jax.numpy as jnp
from jax import lax
from jax.experimental import pallas as pl
from jax.experimental.pallas import tpu as pltpu
from jax.experimental.pallas import tpu_sc as plsc

VOCAB = 1_000_000
D = 64
B_TOTAL = 4096 * 50

_info = plsc.get_sparse_core_info()
NC = _info.num_cores
NS = _info.num_subcores
NW = NC * NS
CH = 128
NCH = B_TOTAL // (NW * CH)
RPW = NCH * CH


def _make_lookup():
  mesh = plsc.VectorSubcoreMesh(core_axis_name="c", subcore_axis_name="s")

  @pl.kernel(
      out_type=jax.ShapeDtypeStruct((B_TOTAL, D), jnp.float32),
      mesh=mesh,
      compiler_params=pltpu.CompilerParams(use_tc_tiling_on_sc=False),
      scratch_types=[
          pltpu.VMEM((NCH, CH), jnp.int32),
          pltpu.VMEM((CH, D), jnp.float32),
          pltpu.SemaphoreType.DMA,
      ],
  )
  def lookup(table_hbm, x_hbm, out_hbm, idx_v, rows_v, sem):
    wid = lax.axis_index("s") * NC + lax.axis_index("c")
    base = wid * RPW
    pltpu.sync_copy(x_hbm.at[wid], idx_v)

    @pl.loop(0, NCH)
    def _chunk(j):
      pltpu.async_copy(table_hbm.at[idx_v.at[j]], rows_v, sem).wait()
      pltpu.sync_copy(rows_v, out_hbm.at[pl.ds(base + j * CH, CH)])

  return lookup


_lookup = _make_lookup()


@jax.jit
def kernel(x, table):
  xf = x.reshape(NW, NCH, CH).astype(jnp.int32)
  out = _lookup(table, xf)
  return out.reshape(x.shape[0], x.shape[1], D)


# speedup vs baseline: 1.0464x; 1.0464x over previous
import jax
import jax.numpy as jnp
from jax import lax
from jax.experimental import pallas as pl
from jax.experimental.pallas import tpu as pltpu
from jax.experimental.pallas import tpu_sc as plsc

VOCAB = 1_000_000
D = 64
B_TOTAL = 4096 * 50

_info = plsc.get_sparse_core_info()
NC = _info.num_cores
NS = _info.num_subcores
NW = NC * NS
CH = 128
NCH = B_TOTAL // (NW * CH)
RPW = NCH * CH
NB = 5


def _make_lookup():
  mesh = plsc.VectorSubcoreMesh(core_axis_name="c", subcore_axis_name="s")

  @pl.kernel(
      out_type=jax.ShapeDtypeStruct((B_TOTAL, D), jnp.float32),
      mesh=mesh,
      compiler_params=pltpu.CompilerParams(use_tc_tiling_on_sc=False),
      scratch_types=(
          [pltpu.VMEM((NCH, CH), jnp.int32)]
          + [pltpu.VMEM((CH, D), jnp.float32) for _ in range(NB)]
          + [pltpu.SemaphoreType.DMA for _ in range(2 * NB)]
      ),
  )
  def lookup(table_hbm, x_hbm, out_hbm, idx_v, *bufs_sems):
    bufs = bufs_sems[:NB]
    sg = bufs_sems[NB:2 * NB]
    sw = bufs_sems[2 * NB:3 * NB]
    wid = lax.axis_index("s") * NC + lax.axis_index("c")
    base = wid * RPW
    pltpu.sync_copy(x_hbm.at[wid], idx_v)

    for b in range(NB):
      pltpu.async_copy(table_hbm.at[idx_v.at[b]], bufs[b], sg[b])

    @pl.loop(0, NCH, step=NB)
    def _chunks(j0):
      for b in range(NB):
        j = j0 + b
        pltpu.make_async_copy(table_hbm.at[idx_v.at[j]], bufs[b], sg[b]).wait()
        pltpu.async_copy(bufs[b], out_hbm.at[pl.ds(base + j * CH, CH)], sw[b])

        @pl.when(j + NB < NCH)
        def _():
          pltpu.make_async_copy(
              bufs[b], out_hbm.at[pl.ds(base + j * CH, CH)], sw[b]).wait()
          pltpu.async_copy(table_hbm.at[idx_v.at[j + NB]], bufs[b], sg[b])

    for b in range(NB):
      j = NCH - NB + b
      pltpu.make_async_copy(
          bufs[b], out_hbm.at[pl.ds(base + j * CH, CH)], sw[b]).wait()

  return lookup


_lookup = _make_lookup()


@jax.jit
def kernel(x, table):
  xf = x.reshape(NW, NCH, CH).astype(jnp.int32)
  out = _lookup(table, xf)
  return out.reshape(x.shape[0], x.shape[1], D)
